# Initial kernel scaffold; baseline (speedup 1.0000x reference)
import jax
import jax.numpy as jnp
from jax.experimental import pallas as pl


def kernel(X_nucl, X_pbs, X_rt, We, Wpbs, Wrt):
    raise NotImplementedError("write your pallas kernel here")


# speedup vs baseline: 6.6580x; 6.6580x over previous
import functools

import jax
import jax.numpy as jnp
from jax import lax
from jax.experimental import pallas as pl
from jax.experimental.pallas import tpu as pltpu
from jax.experimental.pallas import tpu_sc as plsc

EMBED = 128
N_ROWS = 4096 * 200
ROW_GROUPS = N_ROWS // EMBED
NC, NS = 2, 16
NW = NC * NS
PER_W = ROW_GROUPS // NW
CH = 4
NCHUNK = PER_W // CH


def _tab_body(we_ref, wp_ref, wr_ref, out_ref):
    c = lax.broadcasted_iota(jnp.int32, (48, EMBED), 0)
    n = c % 5
    p = (c // 5) % 3
    r = c // 15
    t = jnp.zeros((48, EMBED), jnp.float32)
    for i in range(5):
        t = t + jnp.where(n == i, we_ref[i, :][None, :], 0.0)
    for i in range(3):
        t = t + jnp.where(p == i, wp_ref[i, :][None, :], 0.0)
    for i in range(3):
        t = t + jnp.where(r == i, wr_ref[i, :][None, :], 0.0)
    out_ref[...] = t


def _combined_table(We, Wpbs, Wrt):
    return pl.pallas_call(
        _tab_body,
        out_shape=jax.ShapeDtypeStruct((48, EMBED), jnp.float32),
    )(We, Wpbs, Wrt)


def _sc_embed(tab_hbm, xn_hbm, xp_hbm, xr_hbm, out_hbm,
              xn_v, xp_v, xr_v, xc_v, rows_v, sem):
    wid = lax.axis_index("s") * NC + lax.axis_index("c")

    def chunk(i, carry):
        rbase = wid * PER_W + i * CH
        pltpu.sync_copy(xn_hbm.at[pl.ds(rbase, CH)], xn_v)
        pltpu.sync_copy(xp_hbm.at[pl.ds(rbase, CH)], xp_v)
        pltpu.sync_copy(xr_hbm.at[pl.ds(rbase, CH)], xr_v)
        for j in range(CH):
            for k in range(EMBED // 16):
                s = pl.ds(k * 16, 16)
                xc_v[j, s] = xn_v[j, s] + xp_v[j, s] * 5 + xr_v[j, s] * 15
        cps = [
            pltpu.async_copy(tab_hbm.at[xc_v.at[j]],
                             rows_v.at[pl.ds(j * EMBED, EMBED)], sem)
            for j in range(CH)
        ]
        for cp in cps:
            cp.wait()
        pltpu.sync_copy(rows_v, out_hbm.at[pl.ds(rbase * EMBED, CH * EMBED)])
        return carry

    lax.fori_loop(0, NCHUNK, chunk, 0)


_sc_embed_call = functools.partial(
    pl.kernel,
    out_type=jax.ShapeDtypeStruct((N_ROWS, EMBED), jnp.float32),
    mesh=plsc.VectorSubcoreMesh(core_axis_name="c", subcore_axis_name="s"),
    scratch_types=[
        pltpu.VMEM((CH, EMBED), jnp.int32),
        pltpu.VMEM((CH, EMBED), jnp.int32),
        pltpu.VMEM((CH, EMBED), jnp.int32),
        pltpu.VMEM((CH, EMBED), jnp.int32),
        pltpu.VMEM((CH * EMBED, EMBED), jnp.float32),
        pltpu.SemaphoreType.DMA,
    ],
)(_sc_embed)


@jax.jit
def kernel(X_nucl, X_pbs, X_rt, We, Wpbs, Wrt):
    xn = X_nucl.astype(jnp.int32).reshape(ROW_GROUPS, EMBED)
    xp = X_pbs.astype(jnp.int32).reshape(ROW_GROUPS, EMBED)
    xr = X_rt.astype(jnp.int32).reshape(ROW_GROUPS, EMBED)
    tab = _combined_table(We, Wpbs, Wrt)
    out = _sc_embed_call(tab, xn, xp, xr)
    return out.reshape(X_nucl.shape[0], X_nucl.shape[1], EMBED)


# speedup vs baseline: 6.6628x; 1.0007x over previous
import functools

import jax
import jax.numpy as jnp
from jax import lax
from jax.experimental import pallas as pl
from jax.experimental.pallas import tpu as pltpu
from jax.experimental.pallas import tpu_sc as plsc

EMBED = 128
N_ROWS = 4096 * 200
ROW_GROUPS = N_ROWS // EMBED
NC, NS = 2, 16
NW = NC * NS
PER_W = ROW_GROUPS // NW
HALF = PER_W // 2
CH = 2
NCHUNK = HALF // CH
NSTEP = NCHUNK // 2


def _tab_body(we_ref, wp_ref, wr_ref, out_ref):
    c = lax.broadcasted_iota(jnp.int32, (48, EMBED), 0)
    n = c % 5
    p = (c // 5) % 3
    r = c // 15
    t = jnp.zeros((48, EMBED), jnp.float32)
    for i in range(5):
        t = t + jnp.where(n == i, we_ref[i, :][None, :], 0.0)
    for i in range(3):
        t = t + jnp.where(p == i, wp_ref[i, :][None, :], 0.0)
    for i in range(3):
        t = t + jnp.where(r == i, wr_ref[i, :][None, :], 0.0)
    out_ref[...] = t


def _combined_table(We, Wpbs, Wrt):
    return pl.pallas_call(
        _tab_body,
        out_shape=jax.ShapeDtypeStruct((48, EMBED), jnp.float32),
    )(We, Wpbs, Wrt)


def _sc_embed(tab_hbm, xn_hbm, xp_hbm, xr_hbm, out_hbm,
              xn_v, xp_v, xr_v, rows0, rows1, sg0, sg1, sw0, sw1):
    wid = lax.axis_index("s") * NC + lax.axis_index("c")
    rows = (rows0, rows1)
    sg = (sg0, sg1)
    sw = (sw0, sw1)

    def gather_chunk(i, slot):
        return [
            pltpu.make_async_copy(
                tab_hbm.at[xn_v.at[i * CH + j, 0]],
                rows[slot].at[pl.ds(j * EMBED, EMBED)],
                sg[slot],
            )
            for j in range(CH)
        ]

    def write_chunk(i, slot, hbase):
        return pltpu.make_async_copy(
            rows[slot],
            out_hbm.at[pl.ds((hbase + i * CH) * EMBED, CH * EMBED)],
            sw[slot],
        )

    def half(h, carry):
        hbase = wid * PER_W + h * HALF
        pltpu.sync_copy(xn_hbm.at[pl.ds(hbase, HALF)], xn_v)
        pltpu.sync_copy(xp_hbm.at[pl.ds(hbase, HALF)], xp_v)
        pltpu.sync_copy(xr_hbm.at[pl.ds(hbase, HALF)], xr_v)

        def combine(j, c):
            for k in range(EMBED // 16):
                s = pl.ds(k * 16, 16)
                xn_v[j, 0, s] = xn_v[j, 0, s] + xp_v[j, 0, s] * 5 + xr_v[j, 0, s] * 15
            return c

        lax.fori_loop(0, HALF, combine, 0)

        for s in range(2):
            for cp in gather_chunk(s, s):
                cp.start()

        def step(t, c):
            for s in range(2):
                i = 2 * t + s
                for cp in gather_chunk(i, s):
                    cp.wait()
                write_chunk(i, s, hbase).start()

            @pl.when(t < NSTEP - 1)
            def _prefetch():
                for s in range(2):
                    i = 2 * t + s
                    write_chunk(i, s, hbase).wait()
                    for cp in gather_chunk(i + 2, s):
                        cp.start()

            return c

        lax.fori_loop(0, NSTEP, step, 0)

        for s in range(2):
            write_chunk(NCHUNK - 2 + s, s, hbase).wait()
        return carry

    lax.fori_loop(0, 2, half, 0)


_sc_embed_call = functools.partial(
    pl.kernel,
    out_type=jax.ShapeDtypeStruct((N_ROWS, EMBED), jnp.float32),
    mesh=plsc.VectorSubcoreMesh(core_axis_name="c", subcore_axis_name="s"),
    scratch_types=[
        pltpu.VMEM((HALF, 1, EMBED), jnp.int32),
        pltpu.VMEM((HALF, 1, EMBED), jnp.int32),
        pltpu.VMEM((HALF, 1, EMBED), jnp.int32),
        pltpu.VMEM((CH * EMBED, EMBED), jnp.float32),
        pltpu.VMEM((CH * EMBED, EMBED), jnp.float32),
        pltpu.SemaphoreType.DMA,
        pltpu.SemaphoreType.DMA,
        pltpu.SemaphoreType.DMA,
        pltpu.SemaphoreType.DMA,
    ],
)(_sc_embed)


@jax.jit
def kernel(X_nucl, X_pbs, X_rt, We, Wpbs, Wrt):
    xn = X_nucl.astype(jnp.int32).reshape(ROW_GROUPS, 1, EMBED)
    xp = X_pbs.astype(jnp.int32).reshape(ROW_GROUPS, 1, EMBED)
    xr = X_rt.astype(jnp.int32).reshape(ROW_GROUPS, 1, EMBED)
    tab = _combined_table(We, Wpbs, Wrt)
    out = _sc_embed_call(tab, xn, xp, xr)
    return out.reshape(X_nucl.shape[0], X_nucl.shape[1], EMBED)
